# baseline (speedup 1.0000x reference)
import functools

import jax
import jax.numpy as jnp
from jax import lax
from jax.experimental import pallas as pl
from jax.experimental.pallas import tpu as pltpu
from jax.experimental.pallas import tpu_sc as plsc

_N = 100000
_NSEG = 128
_NW = 16
_CH = 6272
_LAST_CH = _N - (_NW - 1) * _CH
_BINS = _NSEG + 16


def _make_sc_call():
    mesh = plsc.VectorSubcoreMesh(
        core_axis_name="c", subcore_axis_name="s", num_cores=1)

    @functools.partial(
        pl.kernel,
        mesh=mesh,
        out_type=jax.ShapeDtypeStruct((16,), jnp.float32),
        compiler_params=pltpu.CompilerParams(needs_layout_passes=False),
        scratch_types=[
            pltpu.VMEM((_CH,), jnp.float32),
            pltpu.VMEM((_CH,), jnp.float32),
            pltpu.VMEM((_CH,), jnp.int32),
            pltpu.VMEM((_BINS,), jnp.float32),
            pltpu.VMEM((_BINS,), jnp.float32),
            pltpu.VMEM((_NW * 2 * _BINS,), jnp.float32),
            pltpu.VMEM((16,), jnp.float32),
            pltpu.VMEM((16,), jnp.int32),
            pltpu.VMEM_SHARED((_NW * 2 * _BINS,), jnp.float32),
            pltpu.VMEM_SHARED((16,), jnp.float32),
        ],
    )
    def sc_loss(pred_hbm, targ_hbm, batch_hbm, out_hbm,
                pred_v, targ_v, batch_v, sums_v, cnts_v, red_v, out_v,
                ridx_v, stage_sh, red_sh):
        w = lax.axis_index("s")
        base = w * _CH

        zeros16 = jnp.zeros((16,), jnp.float32)
        ones16 = jnp.ones((16,), jnp.float32)
        lane = lax.iota(jnp.int32, 16)

        for j in range(_BINS // 16):
            s = pl.ds(j * 16, 16)
            sums_v[s] = zeros16
            cnts_v[s] = zeros16

        @pl.when(w != _NW - 1)
        def _stage_full():
            pltpu.sync_copy(pred_hbm.at[pl.ds(base, _CH)], pred_v)
            pltpu.sync_copy(targ_hbm.at[pl.ds(base, _CH)], targ_v)
            pltpu.sync_copy(batch_hbm.at[pl.ds(base, _CH)], batch_v)

        @pl.when(w == _NW - 1)
        def _stage_tail():
            tail = pl.ds(0, _LAST_CH)
            pltpu.sync_copy(pred_hbm.at[pl.ds(base, _LAST_CH)], pred_v.at[tail])
            pltpu.sync_copy(targ_hbm.at[pl.ds(base, _LAST_CH)], targ_v.at[tail])
            pltpu.sync_copy(batch_hbm.at[pl.ds(base, _LAST_CH)], batch_v.at[tail])

        def accumulate(n_vregs):
            def body(i, carry):
                s = pl.ds(pl.multiple_of(i * 16, 16), 16)
                p = pred_v[s]
                t = targ_v[s]
                b = batch_v[s]
                v = jnp.abs(p * p - t * t)
                plsc.addupdate_scatter(sums_v, [b], v)
                plsc.addupdate_scatter(cnts_v, [b], ones16)
                return carry
            lax.fori_loop(0, n_vregs, body, 0, unroll=4)

        @pl.when(w != _NW - 1)
        def _accum_full():
            accumulate(_CH // 16)

        @pl.when(w == _NW - 1)
        def _accum_tail():
            accumulate(_LAST_CH // 16)

        woff = w * (2 * _BINS)
        pltpu.sync_copy(sums_v, stage_sh.at[pl.ds(woff, _BINS)])
        pltpu.sync_copy(cnts_v, stage_sh.at[pl.ds(woff + _BINS, _BINS)])
        plsc.subcore_barrier()

        @pl.when(w == _NW - 1)
        def _finalize():
            pltpu.sync_copy(stage_sh, red_v)
            for j in range(_NSEG // 16):
                s = pl.ds(j * 16, 16)
                acc_s = zeros16
                acc_c = zeros16
                for t in range(_NW):
                    toff = t * (2 * _BINS)
                    acc_s = acc_s + red_v[pl.ds(toff + j * 16, 16)]
                    acc_c = acc_c + red_v[pl.ds(toff + _BINS + j * 16, 16)]
                sums_v[s] = acc_s
                cnts_v[s] = acc_c
            last_vec = batch_v[pl.ds(_LAST_CH - 16, 16)]
            max_b = last_vec[15]
            tot = zeros16
            for j in range(_NSEG // 16):
                s = pl.ds(j * 16, 16)
                losses = sums_v[s] / cnts_v[s]
                valid = (lane + (j * 16)) <= max_b
                tot = tot + jnp.where(valid, losses, zeros16)
            out_v[...] = zeros16
            pltpu.sync_copy(out_v, red_sh)
            ridx_v[...] = lane * 0
            out_v[...] = tot
            pltpu.sync_copy(out_v, red_sh.at[ridx_v], add=True)
            pltpu.sync_copy(red_sh, out_v)
            total_vec = zeros16 + out_v[...][0]
            n_graphs = zeros16 + (max_b + 1).astype(jnp.float32)
            out_v[...] = (total_vec / n_graphs) * 10000.0
            pltpu.sync_copy(out_v, out_hbm)

    return sc_loss


_sc_call = _make_sc_call()


@jax.jit
def kernel(pred, target, batch, x):
    del x
    return _sc_call(pred, target, batch)[0]


# speedup vs baseline: 1.3617x; 1.3617x over previous
import functools

import jax
import jax.numpy as jnp
from jax import lax
from jax.experimental import pallas as pl
from jax.experimental.pallas import tpu as pltpu
from jax.experimental.pallas import tpu_sc as plsc

_N = 100000
_NSEG = 128
_NW = 16
_CH = 6272
_LAST_CH = _N - (_NW - 1) * _CH
_BINS = _NSEG + 16


def _make_sc_call():
    mesh = plsc.VectorSubcoreMesh(
        core_axis_name="c", subcore_axis_name="s", num_cores=1)

    @functools.partial(
        pl.kernel,
        mesh=mesh,
        out_type=jax.ShapeDtypeStruct((16,), jnp.float32),
        compiler_params=pltpu.CompilerParams(needs_layout_passes=False),
        scratch_types=[
            pltpu.VMEM((_CH,), jnp.float32),
            pltpu.VMEM((_CH,), jnp.float32),
            pltpu.VMEM((_CH,), jnp.int32),
            pltpu.VMEM((_BINS,), jnp.float32),
            pltpu.VMEM((_BINS,), jnp.float32),
            pltpu.VMEM((_NW * 2 * _BINS,), jnp.float32),
            pltpu.VMEM((16,), jnp.float32),
            pltpu.VMEM((16,), jnp.int32),
            pltpu.VMEM_SHARED((_NW * 2 * _BINS,), jnp.float32),
            pltpu.VMEM_SHARED((16,), jnp.float32),
        ],
    )
    def sc_loss(pred_hbm, targ_hbm, batch_hbm, out_hbm,
                pred_v, targ_v, batch_v, sums_v, cnts_v, red_v, out_v,
                ridx_v, stage_sh, red_sh):
        w = lax.axis_index("s")
        base = w * _CH

        zeros16 = jnp.zeros((16,), jnp.float32)
        ones16 = jnp.ones((16,), jnp.float32)
        lane = lax.iota(jnp.int32, 16)

        for j in range(_BINS // 16):
            s = pl.ds(j * 16, 16)
            sums_v[s] = zeros16
            cnts_v[s] = zeros16

        @pl.when(w != _NW - 1)
        def _stage_full():
            pltpu.sync_copy(pred_hbm.at[pl.ds(base, _CH)], pred_v)
            pltpu.sync_copy(targ_hbm.at[pl.ds(base, _CH)], targ_v)
            pltpu.sync_copy(batch_hbm.at[pl.ds(base, _CH)], batch_v)

        @pl.when(w == _NW - 1)
        def _stage_tail():
            tail = pl.ds(0, _LAST_CH)
            pltpu.sync_copy(pred_hbm.at[pl.ds(base, _LAST_CH)], pred_v.at[tail])
            pltpu.sync_copy(targ_hbm.at[pl.ds(base, _LAST_CH)], targ_v.at[tail])
            pltpu.sync_copy(batch_hbm.at[pl.ds(base, _LAST_CH)], batch_v.at[tail])

        def accumulate(n_vregs):
            lane_base = lane * n_vregs

            def body(i, carry):
                idx = lane_base + i
                p = plsc.load_gather(pred_v, [idx])
                t = plsc.load_gather(targ_v, [idx])
                b = plsc.load_gather(batch_v, [idx])
                v = jnp.abs(p * p - t * t)
                plsc.addupdate_scatter(sums_v, [b], v)
                plsc.addupdate_scatter(cnts_v, [b], ones16)
                return carry
            lax.fori_loop(0, n_vregs, body, 0, unroll=4)

        @pl.when(w != _NW - 1)
        def _accum_full():
            accumulate(_CH // 16)

        @pl.when(w == _NW - 1)
        def _accum_tail():
            accumulate(_LAST_CH // 16)

        woff = w * (2 * _BINS)
        pltpu.sync_copy(sums_v, stage_sh.at[pl.ds(woff, _BINS)])
        pltpu.sync_copy(cnts_v, stage_sh.at[pl.ds(woff + _BINS, _BINS)])
        plsc.subcore_barrier()

        @pl.when(w == _NW - 1)
        def _finalize():
            pltpu.sync_copy(stage_sh, red_v)
            for j in range(_NSEG // 16):
                s = pl.ds(j * 16, 16)
                acc_s = zeros16
                acc_c = zeros16
                for t in range(_NW):
                    toff = t * (2 * _BINS)
                    acc_s = acc_s + red_v[pl.ds(toff + j * 16, 16)]
                    acc_c = acc_c + red_v[pl.ds(toff + _BINS + j * 16, 16)]
                sums_v[s] = acc_s
                cnts_v[s] = acc_c
            last_vec = batch_v[pl.ds(_LAST_CH - 16, 16)]
            max_b = last_vec[15]
            tot = zeros16
            for j in range(_NSEG // 16):
                s = pl.ds(j * 16, 16)
                losses = sums_v[s] / cnts_v[s]
                valid = (lane + (j * 16)) <= max_b
                tot = tot + jnp.where(valid, losses, zeros16)
            out_v[...] = zeros16
            pltpu.sync_copy(out_v, red_sh)
            ridx_v[...] = lane * 0
            out_v[...] = tot
            pltpu.sync_copy(out_v, red_sh.at[ridx_v], add=True)
            pltpu.sync_copy(red_sh, out_v)
            total_vec = zeros16 + out_v[...][0]
            n_graphs = zeros16 + (max_b + 1).astype(jnp.float32)
            out_v[...] = (total_vec / n_graphs) * 10000.0
            pltpu.sync_copy(out_v, out_hbm)

    return sc_loss


_sc_call = _make_sc_call()


@jax.jit
def kernel(pred, target, batch, x):
    del x
    return _sc_call(pred, target, batch)[0]


# speedup vs baseline: 1.4341x; 1.0532x over previous
import functools

import jax
import jax.numpy as jnp
from jax import lax
from jax.experimental import pallas as pl
from jax.experimental.pallas import tpu as pltpu
from jax.experimental.pallas import tpu_sc as plsc

_N = 100000
_NSEG = 128
_NW = 16
_CH = 6272
_LAST_CH = _N - (_NW - 1) * _CH
_BINS = _NSEG + 16


def _make_sc_call():
    mesh = plsc.VectorSubcoreMesh(
        core_axis_name="c", subcore_axis_name="s", num_cores=1)

    @functools.partial(
        pl.kernel,
        mesh=mesh,
        out_type=jax.ShapeDtypeStruct((16,), jnp.float32),
        compiler_params=pltpu.CompilerParams(needs_layout_passes=False),
        scratch_types=[
            pltpu.VMEM((_CH,), jnp.float32),
            pltpu.VMEM((_CH,), jnp.float32),
            pltpu.VMEM((_CH,), jnp.int32),
            pltpu.VMEM((_BINS,), jnp.float32),
            pltpu.VMEM((_BINS,), jnp.float32),
            pltpu.VMEM((_NW * 2 * _BINS,), jnp.float32),
            pltpu.VMEM((16,), jnp.float32),
            pltpu.VMEM_SHARED((_NW * 2 * _BINS,), jnp.float32),
            pltpu.SemaphoreType.DMA,
            pltpu.SemaphoreType.DMA,
            pltpu.SemaphoreType.DMA,
        ],
    )
    def sc_loss(pred_hbm, targ_hbm, batch_hbm, out_hbm,
                pred_v, targ_v, batch_v, sums_v, cnts_v, red_v, out_v,
                stage_sh, sem1, sem2, sem3):
        w = lax.axis_index("s")
        base = w * _CH

        zeros16 = jnp.zeros((16,), jnp.float32)
        ones16 = jnp.ones((16,), jnp.float32)
        lane = lax.iota(jnp.int32, 16)

        for j in range(_BINS // 16):
            s = pl.ds(j * 16, 16)
            sums_v[s] = zeros16
            cnts_v[s] = zeros16

        def stage(n):
            dst = pl.ds(0, n)
            c1 = pltpu.async_copy(pred_hbm.at[pl.ds(base, n)], pred_v.at[dst], sem1)
            c2 = pltpu.async_copy(targ_hbm.at[pl.ds(base, n)], targ_v.at[dst], sem2)
            c3 = pltpu.async_copy(batch_hbm.at[pl.ds(base, n)], batch_v.at[dst], sem3)
            c1.wait()
            c2.wait()
            c3.wait()

        @pl.when(w != _NW - 1)
        def _stage_full():
            stage(_CH)

        @pl.when(w == _NW - 1)
        def _stage_tail():
            stage(_LAST_CH)

        def accumulate(n_vregs):
            lane_base = lane * n_vregs

            def body(i, carry):
                idx = lane_base + i
                p = plsc.load_gather(pred_v, [idx])
                t = plsc.load_gather(targ_v, [idx])
                b = plsc.load_gather(batch_v, [idx])
                v = jnp.abs(p * p - t * t)
                plsc.addupdate_scatter(sums_v, [b], v)
                plsc.addupdate_scatter(cnts_v, [b], ones16)
                return carry
            lax.fori_loop(0, n_vregs, body, 0, unroll=4)

        @pl.when(w != _NW - 1)
        def _accum_full():
            accumulate(_CH // 16)

        @pl.when(w == _NW - 1)
        def _accum_tail():
            accumulate(_LAST_CH // 16)

        woff = w * (2 * _BINS)
        pltpu.sync_copy(sums_v, stage_sh.at[pl.ds(woff, _BINS)])
        pltpu.sync_copy(cnts_v, stage_sh.at[pl.ds(woff + _BINS, _BINS)])
        plsc.subcore_barrier()

        @pl.when(w == _NW - 1)
        def _finalize():
            pltpu.sync_copy(stage_sh, red_v)
            for j in range(_NSEG // 16):
                s = pl.ds(j * 16, 16)
                acc_s = zeros16
                acc_c = zeros16
                for t in range(_NW):
                    toff = t * (2 * _BINS)
                    acc_s = acc_s + red_v[pl.ds(toff + j * 16, 16)]
                    acc_c = acc_c + red_v[pl.ds(toff + _BINS + j * 16, 16)]
                sums_v[s] = acc_s
                cnts_v[s] = acc_c
            last_vec = batch_v[pl.ds(_LAST_CH - 16, 16)]
            max_b = last_vec[15]
            tot = zeros16
            for j in range(_NSEG // 16):
                s = pl.ds(j * 16, 16)
                losses = sums_v[s] / cnts_v[s]
                valid = (lane + (j * 16)) <= max_b
                tot = tot + jnp.where(valid, losses, zeros16)
            total_vec = zeros16 + plsc.cumsum(tot)[15]
            n_graphs = zeros16 + (max_b + 1).astype(jnp.float32)
            out_v[...] = (total_vec / n_graphs) * 10000.0
            pltpu.sync_copy(out_v, out_hbm)

    return sc_loss


_sc_call = _make_sc_call()


@jax.jit
def kernel(pred, target, batch, x):
    del x
    return _sc_call(pred, target, batch)[0]


# speedup vs baseline: 1.4342x; 1.0000x over previous
import functools

import jax
import jax.numpy as jnp
from jax import lax
from jax.experimental import pallas as pl
from jax.experimental.pallas import tpu as pltpu
from jax.experimental.pallas import tpu_sc as plsc

_N = 100000
_NSEG = 128
_NW = 16
_CH = 6272
_LAST_CH = _N - (_NW - 1) * _CH
_BINS = _NSEG + 16


def _make_sc_call():
    mesh = plsc.VectorSubcoreMesh(
        core_axis_name="c", subcore_axis_name="s", num_cores=1)

    @functools.partial(
        pl.kernel,
        mesh=mesh,
        out_type=jax.ShapeDtypeStruct((16,), jnp.float32),
        compiler_params=pltpu.CompilerParams(needs_layout_passes=False, skip_device_barrier=True),
        scratch_types=[
            pltpu.VMEM((_CH,), jnp.float32),
            pltpu.VMEM((_CH,), jnp.float32),
            pltpu.VMEM((_CH,), jnp.int32),
            pltpu.VMEM((_BINS,), jnp.float32),
            pltpu.VMEM((_BINS,), jnp.float32),
            pltpu.VMEM((_NW * 2 * _BINS,), jnp.float32),
            pltpu.VMEM((16,), jnp.float32),
            pltpu.VMEM_SHARED((_NW * 2 * _BINS,), jnp.float32),
            pltpu.SemaphoreType.DMA,
            pltpu.SemaphoreType.DMA,
            pltpu.SemaphoreType.DMA,
        ],
    )
    def sc_loss(pred_hbm, targ_hbm, batch_hbm, out_hbm,
                pred_v, targ_v, batch_v, sums_v, cnts_v, red_v, out_v,
                stage_sh, sem1, sem2, sem3):
        w = lax.axis_index("s")
        base = w * _CH

        zeros16 = jnp.zeros((16,), jnp.float32)
        ones16 = jnp.ones((16,), jnp.float32)
        lane = lax.iota(jnp.int32, 16)

        for j in range(_BINS // 16):
            s = pl.ds(j * 16, 16)
            sums_v[s] = zeros16
            cnts_v[s] = zeros16

        def stage(n):
            dst = pl.ds(0, n)
            c1 = pltpu.async_copy(pred_hbm.at[pl.ds(base, n)], pred_v.at[dst], sem1)
            c2 = pltpu.async_copy(targ_hbm.at[pl.ds(base, n)], targ_v.at[dst], sem2)
            c3 = pltpu.async_copy(batch_hbm.at[pl.ds(base, n)], batch_v.at[dst], sem3)
            c1.wait()
            c2.wait()
            c3.wait()

        @pl.when(w != _NW - 1)
        def _stage_full():
            stage(_CH)

        @pl.when(w == _NW - 1)
        def _stage_tail():
            stage(_LAST_CH)

        def accumulate(n_vregs):
            lane_base = lane * n_vregs

            def body(i, carry):
                idx = lane_base + i
                p = plsc.load_gather(pred_v, [idx])
                t = plsc.load_gather(targ_v, [idx])
                b = plsc.load_gather(batch_v, [idx])
                v = jnp.abs(p * p - t * t)
                plsc.addupdate_scatter(sums_v, [b], v)
                plsc.addupdate_scatter(cnts_v, [b], ones16)
                return carry
            lax.fori_loop(0, n_vregs, body, 0, unroll=4)

        @pl.when(w != _NW - 1)
        def _accum_full():
            accumulate(_CH // 16)

        @pl.when(w == _NW - 1)
        def _accum_tail():
            accumulate(_LAST_CH // 16)

        woff = w * (2 * _BINS)
        pltpu.sync_copy(sums_v, stage_sh.at[pl.ds(woff, _BINS)])
        pltpu.sync_copy(cnts_v, stage_sh.at[pl.ds(woff + _BINS, _BINS)])
        plsc.subcore_barrier()

        @pl.when(w == _NW - 1)
        def _finalize():
            pltpu.sync_copy(stage_sh, red_v)
            for j in range(_NSEG // 16):
                s = pl.ds(j * 16, 16)
                acc_s = zeros16
                acc_c = zeros16
                for t in range(_NW):
                    toff = t * (2 * _BINS)
                    acc_s = acc_s + red_v[pl.ds(toff + j * 16, 16)]
                    acc_c = acc_c + red_v[pl.ds(toff + _BINS + j * 16, 16)]
                sums_v[s] = acc_s
                cnts_v[s] = acc_c
            last_vec = batch_v[pl.ds(_LAST_CH - 16, 16)]
            max_b = last_vec[15]
            tot = zeros16
            for j in range(_NSEG // 16):
                s = pl.ds(j * 16, 16)
                losses = sums_v[s] / cnts_v[s]
                valid = (lane + (j * 16)) <= max_b
                tot = tot + jnp.where(valid, losses, zeros16)
            total_vec = zeros16 + plsc.cumsum(tot)[15]
            n_graphs = zeros16 + (max_b + 1).astype(jnp.float32)
            out_v[...] = (total_vec / n_graphs) * 10000.0
            pltpu.sync_copy(out_v, out_hbm)

    return sc_loss


_sc_call = _make_sc_call()


@jax.jit
def kernel(pred, target, batch, x):
    del x
    return _sc_call(pred, target, batch)[0]
